# baseline (speedup 1.0000x reference)
import functools

import jax
import jax.numpy as jnp
from jax import lax
from jax.experimental import pallas as pl
from jax.experimental.pallas import tpu as pltpu
from jax.experimental.pallas import tpu_sc as plsc

NUM_ROWS = 129
EMBED_D = 64
BATCH = 4096
HIST = 200

_NC = 2
_NS = 16
_NW = _NC * _NS
_BPW = BATCH // _NW
_L = 16
_DP = EMBED_D // 2
_TSTRIDE = 144
_HC = 8
_NHC = HIST // _HC

_mesh = plsc.VectorSubcoreMesh(core_axis_name="c", subcore_axis_name="s")


@functools.partial(
    pl.kernel,
    mesh=_mesh,
    compiler_params=pltpu.CompilerParams(needs_layout_passes=False),
    out_type=jax.ShapeDtypeStruct((HIST, EMBED_D, BATCH), jnp.float32),
    scratch_types=[
        pltpu.VMEM((_DP, _TSTRIDE), jnp.int32),
        pltpu.VMEM((4 * _DP * _TSTRIDE,), jnp.int32),
        pltpu.VMEM((_HC, _BPW), jnp.int32),
        pltpu.VMEM((_HC, _BPW), jnp.int32),
        pltpu.VMEM((EMBED_D, _BPW), jnp.float32),
        pltpu.VMEM((EMBED_D, _BPW), jnp.float32),
        pltpu.SemaphoreType.DMA,
        pltpu.SemaphoreType.DMA,
        pltpu.SemaphoreType.DMA,
        pltpu.SemaphoreType.DMA,
    ],
)
def _embed_gather(idx_hbm, table_hbm, out_hbm, table2d, table_f, idx0, idx1,
                  slab0, slab1, si0, si1, so0, so1):
    wid = lax.axis_index("s") * _NC + lax.axis_index("c")
    b0 = wid * _BPW

    pltpu.sync_copy(table_hbm, table2d)
    pltpu.async_copy(idx_hbm.at[pl.ds(0, _HC), pl.ds(b0, _BPW)], idx0, si0)
    pltpu.async_copy(idx_hbm.at[pl.ds(_HC, _HC), pl.ds(b0, _BPW)], idx1, si1)

    lanes4 = lax.iota(jnp.int32, _L) * 4

    def flat(p, carry):
        for k in range(_TSTRIDE // _L):
            v = table2d[p, pl.ds(k * _L, _L)]
            base = (p * _TSTRIDE + k * _L) * 4
            for r in range(4):
                plsc.store_scatter(table_f, [lanes4 + (base + r)], v)
        return carry

    lax.fori_loop(0, _DP, flat, 0)

    def slab_compute(iv_ref, hh, rv):
        lmod = lax.rem(lax.iota(jnp.int32, _L), 4)
        for s in range(_BPW // _L):
            g4 = iv_ref[hh, pl.ds(s * _L, _L)] * 4 + lmod
            for p0 in range(0, _DP, 8):
                ws = [
                    plsc.load_gather(
                        table_f, [g4 + ((p0 + u) * _TSTRIDE * 4)])
                    for u in range(8)
                ]
                for u in range(8):
                    bf = plsc.bitcast(ws[u], jnp.bfloat16)
                    lo, hi = plsc.unpack(
                        bf, format=plsc.PackFormat.INTERLEAVED)
                    rv[2 * (p0 + u), pl.ds(s * _L, _L)] = lo
                    rv[2 * (p0 + u) + 1, pl.ds(s * _L, _L)] = hi

    def chunk(hc, carry):
        for p, (iv_ref, si) in enumerate(((idx0, si0), (idx1, si1))):
            @pl.when(hc % 2 == p)
            def _run():
                pltpu.make_async_copy(
                    idx_hbm.at[pl.ds(0, _HC), pl.ds(b0, _BPW)], iv_ref,
                    si).wait()

                def hpair(q, carry2):
                    for par, (rv, so) in enumerate(((slab0, so0),
                                                    (slab1, so1))):
                        hh = 2 * q + par
                        h = hc * _HC + hh

                        @pl.when(h >= 2)
                        def _wait_out():
                            pltpu.make_async_copy(
                                rv, out_hbm.at[0, :, pl.ds(b0, _BPW)],
                                so).wait()

                        slab_compute(iv_ref, hh, rv)
                        pltpu.async_copy(
                            rv, out_hbm.at[h, :, pl.ds(b0, _BPW)], so)
                    return carry2

                lax.fori_loop(0, _HC // 2, hpair, 0)

                @pl.when(hc + 2 < _NHC)
                def _prefetch():
                    pltpu.async_copy(
                        idx_hbm.at[pl.ds((hc + 2) * _HC, _HC),
                                   pl.ds(b0, _BPW)], iv_ref, si)
        return carry

    lax.fori_loop(0, _NHC, chunk, 0)

    pltpu.make_async_copy(slab0, out_hbm.at[0, :, pl.ds(b0, _BPW)], so0).wait()
    pltpu.make_async_copy(slab1, out_hbm.at[0, :, pl.ds(b0, _BPW)], so1).wait()


def _pack_table(table):
    tb = table.astype(jnp.bfloat16)
    u = lax.bitcast_convert_type(tb, jnp.uint16)
    lo = u[:, 0::2].astype(jnp.uint32)
    hi = u[:, 1::2].astype(jnp.uint32)
    packed = lax.bitcast_convert_type(lo | (hi << 16), jnp.int32)
    tp = packed.T
    return jnp.pad(tp, ((0, 0), (0, _TSTRIDE - NUM_ROWS)))


def kernel(genres, table):
    gt = genres.astype(jnp.int32).T
    out_t = _embed_gather(gt, _pack_table(table))
    return out_t.transpose(2, 0, 1)


# speedup vs baseline: 1.0261x; 1.0261x over previous
import functools

import jax
import jax.numpy as jnp
from jax import lax
from jax.experimental import pallas as pl
from jax.experimental.pallas import tpu as pltpu
from jax.experimental.pallas import tpu_sc as plsc

NUM_ROWS = 129
EMBED_D = 64
BATCH = 4096
HIST = 200

_NC = 2
_NS = 16
_NW = _NC * _NS
_BPW = BATCH // _NW
_L = 16
_DP = EMBED_D // 2
_TSTRIDE = 144
_HC = 8
_NHC = HIST // _HC

_mesh = plsc.VectorSubcoreMesh(core_axis_name="c", subcore_axis_name="s")


@functools.partial(
    pl.kernel,
    mesh=_mesh,
    compiler_params=pltpu.CompilerParams(needs_layout_passes=False),
    out_type=jax.ShapeDtypeStruct((HIST, EMBED_D, BATCH), jnp.float32),
    scratch_types=[
        pltpu.VMEM((_DP, _TSTRIDE), jnp.int32),
        pltpu.VMEM((_DP * _TSTRIDE,), jnp.int32),
        pltpu.VMEM((_HC, _BPW), jnp.int32),
        pltpu.VMEM((_HC, _BPW), jnp.int32),
        pltpu.VMEM((EMBED_D, _BPW), jnp.float32),
        pltpu.VMEM((EMBED_D, _BPW), jnp.float32),
        pltpu.SemaphoreType.DMA,
        pltpu.SemaphoreType.DMA,
        pltpu.SemaphoreType.DMA,
        pltpu.SemaphoreType.DMA,
    ],
)
def _embed_gather(idx_hbm, table_hbm, out_hbm, table2d, table_f, idx0, idx1,
                  slab0, slab1, si0, si1, so0, so1):
    wid = lax.axis_index("s") * _NC + lax.axis_index("c")
    b0 = wid * _BPW

    pltpu.sync_copy(table_hbm, table2d)
    pltpu.async_copy(idx_hbm.at[pl.ds(0, _HC), pl.ds(b0, _BPW)], idx0, si0)
    pltpu.async_copy(idx_hbm.at[pl.ds(_HC, _HC), pl.ds(b0, _BPW)], idx1, si1)

    def flat(p, carry):
        for k in range(_TSTRIDE // _L):
            v = table2d[p, pl.ds(k * _L, _L)]
            table_f[pl.ds(p * _TSTRIDE + k * _L, _L)] = v
        return carry

    lax.fori_loop(0, _DP, flat, 0)

    def slab_compute(iv_ref, hh, rv):
        for s in range(_BPW // _L):
            g = iv_ref[hh, pl.ds(s * _L, _L)]
            for p0 in range(0, _DP, 8):
                ws = [
                    plsc.load_gather(table_f, [g + ((p0 + u) * _TSTRIDE)])
                    for u in range(8)
                ]
                for u in range(8):
                    bf = plsc.bitcast(ws[u], jnp.bfloat16)
                    lo, hi = plsc.unpack(
                        bf, format=plsc.PackFormat.INTERLEAVED)
                    rv[2 * (p0 + u), pl.ds(s * _L, _L)] = lo
                    rv[2 * (p0 + u) + 1, pl.ds(s * _L, _L)] = hi

    def chunk(hc, carry):
        for p, (iv_ref, si) in enumerate(((idx0, si0), (idx1, si1))):
            @pl.when(hc % 2 == p)
            def _run():
                pltpu.make_async_copy(
                    idx_hbm.at[pl.ds(0, _HC), pl.ds(b0, _BPW)], iv_ref,
                    si).wait()

                def hpair(q, carry2):
                    for par, (rv, so) in enumerate(((slab0, so0),
                                                    (slab1, so1))):
                        hh = 2 * q + par
                        h = hc * _HC + hh

                        @pl.when(h >= 2)
                        def _wait_out():
                            pltpu.make_async_copy(
                                rv, out_hbm.at[0, :, pl.ds(b0, _BPW)],
                                so).wait()

                        slab_compute(iv_ref, hh, rv)
                        pltpu.async_copy(
                            rv, out_hbm.at[h, :, pl.ds(b0, _BPW)], so)
                    return carry2

                lax.fori_loop(0, _HC // 2, hpair, 0)

                @pl.when(hc + 2 < _NHC)
                def _prefetch():
                    pltpu.async_copy(
                        idx_hbm.at[pl.ds((hc + 2) * _HC, _HC),
                                   pl.ds(b0, _BPW)], iv_ref, si)
        return carry

    lax.fori_loop(0, _NHC, chunk, 0)

    pltpu.make_async_copy(slab0, out_hbm.at[0, :, pl.ds(b0, _BPW)], so0).wait()
    pltpu.make_async_copy(slab1, out_hbm.at[0, :, pl.ds(b0, _BPW)], so1).wait()


def _pack_table(table):
    tb = table.astype(jnp.bfloat16)
    u = lax.bitcast_convert_type(tb, jnp.uint16)
    lo = u[:, 0::2].astype(jnp.uint32)
    hi = u[:, 1::2].astype(jnp.uint32)
    packed = lax.bitcast_convert_type(lo | (hi << 16), jnp.int32)
    tp = packed.T
    return jnp.pad(tp, ((0, 0), (0, _TSTRIDE - NUM_ROWS)))


def kernel(genres, table):
    gt = genres.astype(jnp.int32).T
    out_t = _embed_gather(gt, _pack_table(table))
    return out_t.transpose(2, 0, 1)
